# baseline (speedup 1.0000x reference)
import functools

import jax
import jax.numpy as jnp
from jax import lax
from jax.experimental import pallas as pl
from jax.experimental.pallas import tpu as pltpu
from jax.experimental.pallas import tpu_sc as plsc

MAXLEN = 512
PE_ROWS = 2 * MAXLEN + 1
PAD_ROWS = 1032
NUM_TABLES = 4
CHUNK = 64
NBUF = 4


def _table_kernel(pe_ref, w_ref, b_ref, out_ref):
    acc = lax.dot_general(
        pe_ref[...], w_ref[...],
        (((1,), (1,)), ((), ())),
        preferred_element_type=jnp.float32,
    )
    out_ref[0] = acc + 0.25 * b_ref[...]


def _idx_kernel(ps_ref, pe_ref, out_ref, *, seq):
    k = pl.program_id(0)
    s = ps_ref[0, 0]
    e = pe_ref[0, 0]
    row = jnp.where(k < 2, s, e)
    col = jnp.where(k % 2 == 0, s, e)
    base = MAXLEN + k * PAD_ROWS
    row2 = lax.broadcast_in_dim(row, (seq, seq), (0,))
    col2 = lax.broadcast_in_dim(col, (seq, seq), (1,))
    out_ref[0, 0] = row2 - col2 + base


def _make_sc_gather(n_rows, hidden, num_workers, num_cores):
    chunks_per_worker = n_rows // (num_workers * CHUNK)
    n_chunks = n_rows // CHUNK
    mesh = plsc.VectorSubcoreMesh(core_axis_name="c", subcore_axis_name="s")

    @functools.partial(
        pl.kernel,
        mesh=mesh,
        compiler_params=pltpu.CompilerParams(use_tc_tiling_on_sc=False),
        out_type=jax.ShapeDtypeStruct((n_chunks, CHUNK, hidden), jnp.float32),
        scratch_types=(
            [pltpu.VMEM((NUM_TABLES, CHUNK), jnp.int32)] * NBUF
            + [pltpu.VMEM((NUM_TABLES, CHUNK, hidden // 2), jnp.int32)] * NBUF
            + [pltpu.VMEM((CHUNK, hidden), jnp.float32)] * 2
            + [pltpu.SemaphoreType.DMA] * (2 * NBUF + 2)
        ),
    )
    def sc_gather(table_hbm, idx_hbm, out_hbm, *scratch):
        ix = scratch[:NBUF]
        gb = scratch[NBUF:2 * NBUF]
        ob = scratch[2 * NBUF:2 * NBUF + 2]
        semi = scratch[2 * NBUF + 2:3 * NBUF + 2]
        semg = scratch[3 * NBUF + 2:4 * NBUF + 2]
        semo = scratch[4 * NBUF + 2:]
        wid = lax.axis_index("s") * num_cores + lax.axis_index("c")
        chunk0 = wid * chunks_per_worker
        last = chunks_per_worker - 1

        def issue_idx(c, p):
            ch = chunk0 + jnp.minimum(c, last)
            pltpu.async_copy(idx_hbm.at[ch], ix[p], semi[p])

        def wait_idx(p):
            pltpu.make_async_copy(idx_hbm.at[chunk0], ix[p], semi[p]).wait()

        def issue_gathers(p):
            for k in range(NUM_TABLES):
                pltpu.async_copy(table_hbm.at[ix[p].at[k]], gb[p].at[k],
                                 semg[p])

        def wait_gathers(p):
            for k in range(NUM_TABLES):
                pltpu.make_async_copy(table_hbm.at[ix[p].at[k]], gb[p].at[k],
                                      semg[p]).wait()

        def wait_store(p):
            pltpu.make_async_copy(ob[p], out_hbm.at[chunk0], semo[p]).wait()

        def compute_store(c, p, po):
            gbp = gb[p]
            obp = ob[po]

            def lo_f32(w):
                return lax.bitcast_convert_type(w << 16, jnp.float32)

            def hi_f32(w):
                return lax.bitcast_convert_type(w & jnp.int32(-65536),
                                                jnp.float32)

            @plsc.parallel_loop(0, CHUNK, unroll=2)
            def row_body(r):
                for g in range(hidden // 32):
                    sl = pl.ds(g * 16, 16)
                    w0 = gbp[0, r, sl]
                    w1 = gbp[1, r, sl]
                    w2 = gbp[2, r, sl]
                    w3 = gbp[3, r, sl]
                    lo = lo_f32(w0) + lo_f32(w1) + lo_f32(w2) + lo_f32(w3)
                    hi = hi_f32(w0) + hi_f32(w1) + hi_f32(w2) + hi_f32(w3)
                    obp[r, pl.ds(g * 32, 16)] = jnp.maximum(lo, 0.0)
                    obp[r, pl.ds(g * 32 + 16, 16)] = jnp.maximum(hi, 0.0)
            pltpu.async_copy(obp, out_hbm.at[chunk0 + c], semo[po])

        for q in range(NBUF):
            issue_idx(q, q)
        for q in range(NBUF - 1):
            wait_idx(q)
            issue_gathers(q)

        def ring_body(c4, carry):
            for p in range(NBUF):
                c = NBUF * c4 + p
                po = p % 2
                prev = (p + NBUF - 1) % NBUF
                wait_gathers(p)
                issue_idx(c + NBUF, p)
                wait_idx(prev)
                issue_gathers(prev)
                @pl.when(c >= 2)
                def _():
                    wait_store(po)
                compute_store(c, p, po)
            return carry

        lax.fori_loop(0, chunks_per_worker // NBUF, ring_body, 0)

        p_last = (chunks_per_worker - 1) % NBUF
        wait_idx(p_last)
        for q in range(NBUF):
            if q != p_last:
                wait_gathers(q)
        wait_store(0)
        wait_store(1)

    return sc_gather


def kernel(pos_start, pos_end, pe, W, b):
    B, S = pos_start.shape
    H = pe.shape[1]
    n_rows = B * S * S

    pe_pad = jnp.pad(pe, ((0, PAD_ROWS - pe.shape[0]), (0, 0)))

    table = pl.pallas_call(
        _table_kernel,
        grid=(NUM_TABLES,),
        in_specs=[
            pl.BlockSpec((PAD_ROWS, H), lambda k: (0, 0)),
            pl.BlockSpec((H, H), lambda k: (0, k)),
            pl.BlockSpec((1, H), lambda k: (0, 0)),
        ],
        out_specs=pl.BlockSpec((1, PAD_ROWS, H), lambda k: (k, 0, 0)),
        out_shape=jax.ShapeDtypeStruct((NUM_TABLES, PAD_ROWS, H), jnp.float32),
    )(pe_pad, W, b.reshape(1, H))

    idx = pl.pallas_call(
        functools.partial(_idx_kernel, seq=S),
        grid=(NUM_TABLES, B),
        in_specs=[
            pl.BlockSpec((1, 1, S), lambda k, bb: (bb, 0, 0)),
            pl.BlockSpec((1, 1, S), lambda k, bb: (bb, 0, 0)),
        ],
        out_specs=pl.BlockSpec((1, 1, S, S), lambda k, bb: (k, bb, 0, 0)),
        out_shape=jax.ShapeDtypeStruct((NUM_TABLES, B, S, S), jnp.int32),
    )(pos_start.reshape(B, 1, S), pos_end.reshape(B, 1, S))

    info = plsc.get_sparse_core_info()
    num_workers = info.num_cores * info.num_subcores
    sc_gather = _make_sc_gather(n_rows, H, num_workers, info.num_cores)
    perm = (jnp.arange(H) // 32) * 32 + jnp.where(
        jnp.arange(H) % 2 == 0, (jnp.arange(H) % 32) // 2,
        (jnp.arange(H) % 32) // 2 + 16)
    table_bf = table[:, :, perm].astype(jnp.bfloat16)
    table_i32 = jax.lax.bitcast_convert_type(
        table_bf.reshape(NUM_TABLES * PAD_ROWS, H // 2, 2), jnp.int32)

    out = sc_gather(
        table_i32,
        idx.reshape(NUM_TABLES, n_rows // CHUNK, CHUNK).transpose(1, 0, 2),
    )
    return out.reshape(B, S, S, H)


# speedup vs baseline: 1.0520x; 1.0520x over previous
import functools

import jax
import jax.numpy as jnp
from jax import lax
from jax.experimental import pallas as pl
from jax.experimental.pallas import tpu as pltpu
from jax.experimental.pallas import tpu_sc as plsc

MAXLEN = 512
PE_ROWS = 2 * MAXLEN + 1
PAD_ROWS = 1032
NUM_TABLES = 4
CHUNK = 128


def _table_kernel(pe_ref, w_ref, b_ref, out_ref):
    acc = lax.dot_general(
        pe_ref[...], w_ref[...],
        (((1,), (1,)), ((), ())),
        preferred_element_type=jnp.float32,
    )
    out_ref[0] = acc + 0.25 * b_ref[...]


def _idx_kernel(ps_ref, pe_ref, out_ref, *, seq):
    k = pl.program_id(0)
    s = ps_ref[0, 0]
    e = pe_ref[0, 0]
    row = jnp.where(k < 2, s, e)
    col = jnp.where(k % 2 == 0, s, e)
    base = MAXLEN + k * PAD_ROWS
    row2 = lax.broadcast_in_dim(row, (seq, seq), (0,))
    col2 = lax.broadcast_in_dim(col, (seq, seq), (1,))
    out_ref[0, 0] = row2 - col2 + base


def _make_sc_gather(n_rows, hidden, num_workers, num_cores):
    chunks_per_worker = n_rows // (num_workers * CHUNK)
    n_chunks = n_rows // CHUNK
    mesh = plsc.VectorSubcoreMesh(core_axis_name="c", subcore_axis_name="s")

    @functools.partial(
        pl.kernel,
        mesh=mesh,
        compiler_params=pltpu.CompilerParams(use_tc_tiling_on_sc=False),
        out_type=jax.ShapeDtypeStruct((n_chunks, CHUNK, hidden), jnp.float32),
        scratch_types=[
            pltpu.VMEM((NUM_TABLES, CHUNK), jnp.int32),
            pltpu.VMEM((NUM_TABLES, CHUNK), jnp.int32),
            pltpu.VMEM((NUM_TABLES, CHUNK, hidden // 2), jnp.int32),
            pltpu.VMEM((NUM_TABLES, CHUNK, hidden // 2), jnp.int32),
            pltpu.VMEM((CHUNK, hidden), jnp.float32),
            pltpu.VMEM((CHUNK, hidden), jnp.float32),
            pltpu.SemaphoreType.DMA,
            pltpu.SemaphoreType.DMA,
            pltpu.SemaphoreType.DMA,
            pltpu.SemaphoreType.DMA,
            pltpu.SemaphoreType.DMA,
            pltpu.SemaphoreType.DMA,
        ],
    )
    def sc_gather(table_hbm, idx_hbm, out_hbm,
                  ixA, ixB, gbA, gbB, obA, obB,
                  semiA, semiB, semgA, semgB, semoA, semoB):
        wid = lax.axis_index("s") * num_cores + lax.axis_index("c")
        chunk0 = wid * chunks_per_worker
        last = chunks_per_worker - 1

        ix = (ixA, ixB)
        gb = (gbA, gbB)
        ob = (obA, obB)
        semi = (semiA, semiB)
        semg = (semgA, semgB)
        semo = (semoA, semoB)

        def issue_idx(c, p):
            ch = chunk0 + jnp.minimum(c, last)
            pltpu.async_copy(idx_hbm.at[ch], ix[p], semi[p])

        def wait_idx(p):
            pltpu.make_async_copy(idx_hbm.at[chunk0], ix[p], semi[p]).wait()

        def issue_gathers(p):
            for k in range(NUM_TABLES):
                pltpu.async_copy(table_hbm.at[ix[p].at[k]], gb[p].at[k],
                                 semg[p])

        def wait_gathers(p):
            for k in range(NUM_TABLES):
                pltpu.make_async_copy(table_hbm.at[ix[p].at[k]], gb[p].at[k],
                                      semg[p]).wait()

        def wait_store(p):
            pltpu.make_async_copy(ob[p], out_hbm.at[chunk0], semo[p]).wait()

        def compute_store(c, p):
            gbp = gb[p]
            obp = ob[p]

            def lo_f32(w):
                return lax.bitcast_convert_type(w << 16, jnp.float32)

            def hi_f32(w):
                return lax.bitcast_convert_type(w & jnp.int32(-65536),
                                                jnp.float32)

            @plsc.parallel_loop(0, CHUNK, unroll=2)
            def row_body(r):
                for g in range(hidden // 32):
                    sl = pl.ds(g * 16, 16)
                    w0 = gbp[0, r, sl]
                    w1 = gbp[1, r, sl]
                    w2 = gbp[2, r, sl]
                    w3 = gbp[3, r, sl]
                    lo = lo_f32(w0) + lo_f32(w1) + lo_f32(w2) + lo_f32(w3)
                    hi = hi_f32(w0) + hi_f32(w1) + hi_f32(w2) + hi_f32(w3)
                    obp[r, pl.ds(g * 32, 16)] = jnp.maximum(lo, 0.0)
                    obp[r, pl.ds(g * 32 + 16, 16)] = jnp.maximum(hi, 0.0)
            pltpu.async_copy(obp, out_hbm.at[chunk0 + c], semo[p])

        issue_idx(0, 0)
        issue_idx(1, 1)
        wait_idx(0)
        issue_gathers(0)

        def pair_body(c2, carry):
            for p in (0, 1):
                c = 2 * c2 + p
                o = 1 - p
                wait_gathers(p)
                issue_idx(c + 2, p)
                wait_idx(o)
                issue_gathers(o)
                @pl.when(c >= 2)
                def _():
                    wait_store(p)
                compute_store(c, p)
            return carry

        lax.fori_loop(0, chunks_per_worker // 2, pair_body, 0)

        wait_gathers(0)
        wait_idx(1)
        wait_store(0)
        wait_store(1)

    return sc_gather


def kernel(pos_start, pos_end, pe, W, b):
    B, S = pos_start.shape
    H = pe.shape[1]
    n_rows = B * S * S

    pe_pad = jnp.pad(pe, ((0, PAD_ROWS - pe.shape[0]), (0, 0)))

    table = pl.pallas_call(
        _table_kernel,
        grid=(NUM_TABLES,),
        in_specs=[
            pl.BlockSpec((PAD_ROWS, H), lambda k: (0, 0)),
            pl.BlockSpec((H, H), lambda k: (0, k)),
            pl.BlockSpec((1, H), lambda k: (0, 0)),
        ],
        out_specs=pl.BlockSpec((1, PAD_ROWS, H), lambda k: (k, 0, 0)),
        out_shape=jax.ShapeDtypeStruct((NUM_TABLES, PAD_ROWS, H), jnp.float32),
    )(pe_pad, W, b.reshape(1, H))

    idx = pl.pallas_call(
        functools.partial(_idx_kernel, seq=S),
        grid=(NUM_TABLES, B),
        in_specs=[
            pl.BlockSpec((1, 1, S), lambda k, bb: (bb, 0, 0)),
            pl.BlockSpec((1, 1, S), lambda k, bb: (bb, 0, 0)),
        ],
        out_specs=pl.BlockSpec((1, 1, S, S), lambda k, bb: (k, bb, 0, 0)),
        out_shape=jax.ShapeDtypeStruct((NUM_TABLES, B, S, S), jnp.int32),
    )(pos_start.reshape(B, 1, S), pos_end.reshape(B, 1, S))

    info = plsc.get_sparse_core_info()
    num_workers = info.num_cores * info.num_subcores
    sc_gather = _make_sc_gather(n_rows, H, num_workers, info.num_cores)
    perm = (jnp.arange(H) // 32) * 32 + jnp.where(
        jnp.arange(H) % 2 == 0, (jnp.arange(H) % 32) // 2,
        (jnp.arange(H) % 32) // 2 + 16)
    table_bf = table[:, :, perm].astype(jnp.bfloat16)
    table_i32 = jax.lax.bitcast_convert_type(
        table_bf.reshape(NUM_TABLES * PAD_ROWS, H // 2, 2), jnp.int32)

    out = sc_gather(
        table_i32,
        idx.reshape(NUM_TABLES, n_rows // CHUNK, CHUNK).transpose(1, 0, 2),
    )
    return out.reshape(B, S, S, H)


# speedup vs baseline: 1.0652x; 1.0125x over previous
import functools

import jax
import jax.numpy as jnp
from jax import lax
from jax.experimental import pallas as pl
from jax.experimental.pallas import tpu as pltpu
from jax.experimental.pallas import tpu_sc as plsc

MAXLEN = 512
PE_ROWS = 2 * MAXLEN + 1
PAD_ROWS = 1032
NUM_TABLES = 4
CHUNK = 128


def _table_kernel(pe_ref, w_ref, b_ref, out_ref):
    acc = lax.dot_general(
        pe_ref[...], w_ref[...],
        (((1,), (1,)), ((), ())),
        preferred_element_type=jnp.float32,
    )
    out_ref[0] = (acc + 0.25 * b_ref[...]).astype(jnp.bfloat16)


def _idx_kernel(ps_ref, pe_ref, out_ref, *, seq):
    k = pl.program_id(0)
    s = ps_ref[0, 0]
    e = pe_ref[0, 0]
    row = jnp.where(k < 2, s, e)
    col = jnp.where(k % 2 == 0, s, e)
    base = MAXLEN + k * PAD_ROWS
    row2 = lax.broadcast_in_dim(row, (seq, seq), (0,))
    col2 = lax.broadcast_in_dim(col, (seq, seq), (1,))
    out_ref[0, 0] = row2 - col2 + base


def _make_sc_gather(n_rows, hidden, num_workers, num_cores):
    chunks_per_worker = n_rows // (num_workers * CHUNK)
    n_chunks = n_rows // CHUNK
    mesh = plsc.VectorSubcoreMesh(core_axis_name="c", subcore_axis_name="s")

    @functools.partial(
        pl.kernel,
        mesh=mesh,
        compiler_params=pltpu.CompilerParams(use_tc_tiling_on_sc=False),
        out_type=jax.ShapeDtypeStruct((n_chunks, CHUNK, hidden), jnp.float32),
        scratch_types=[
            pltpu.VMEM((NUM_TABLES, CHUNK), jnp.int32),
            pltpu.VMEM((NUM_TABLES, CHUNK), jnp.int32),
            pltpu.VMEM((NUM_TABLES, CHUNK, hidden // 2), jnp.int32),
            pltpu.VMEM((NUM_TABLES, CHUNK, hidden // 2), jnp.int32),
            pltpu.VMEM((CHUNK, hidden), jnp.float32),
            pltpu.VMEM((CHUNK, hidden), jnp.float32),
            pltpu.SemaphoreType.DMA,
            pltpu.SemaphoreType.DMA,
            pltpu.SemaphoreType.DMA,
            pltpu.SemaphoreType.DMA,
            pltpu.SemaphoreType.DMA,
            pltpu.SemaphoreType.DMA,
        ],
    )
    def sc_gather(table_hbm, idx_hbm, out_hbm,
                  ixA, ixB, gbA, gbB, obA, obB,
                  semiA, semiB, semgA, semgB, semoA, semoB):
        wid = lax.axis_index("s") * num_cores + lax.axis_index("c")
        chunk0 = wid * chunks_per_worker
        last = chunks_per_worker - 1

        ix = (ixA, ixB)
        gb = (gbA, gbB)
        ob = (obA, obB)
        semi = (semiA, semiB)
        semg = (semgA, semgB)
        semo = (semoA, semoB)

        def issue_idx(c, p):
            ch = chunk0 + jnp.minimum(c, last)
            pltpu.async_copy(idx_hbm.at[:, ch], ix[p], semi[p])

        def wait_idx(p):
            pltpu.make_async_copy(idx_hbm.at[:, chunk0], ix[p],
                                  semi[p]).wait()

        def issue_gathers(p):
            for k in range(NUM_TABLES):
                pltpu.async_copy(table_hbm.at[ix[p].at[k]], gb[p].at[k],
                                 semg[p])

        def wait_gathers(p):
            for k in range(NUM_TABLES):
                pltpu.make_async_copy(table_hbm.at[ix[p].at[k]], gb[p].at[k],
                                      semg[p]).wait()

        def wait_store(p):
            pltpu.make_async_copy(ob[p], out_hbm.at[chunk0], semo[p]).wait()

        def compute_store(c, p):
            gbp = gb[p]
            obp = ob[p]

            def lo_f32(w):
                return lax.bitcast_convert_type(w << 16, jnp.float32)

            def hi_f32(w):
                return lax.bitcast_convert_type(w & jnp.int32(-65536),
                                                jnp.float32)

            @plsc.parallel_loop(0, CHUNK, unroll=2)
            def row_body(r):
                for g in range(hidden // 32):
                    sl = pl.ds(g * 16, 16)
                    w0 = gbp[0, r, sl]
                    w1 = gbp[1, r, sl]
                    w2 = gbp[2, r, sl]
                    w3 = gbp[3, r, sl]
                    lo = lo_f32(w0) + lo_f32(w1) + lo_f32(w2) + lo_f32(w3)
                    hi = hi_f32(w0) + hi_f32(w1) + hi_f32(w2) + hi_f32(w3)
                    obp[r, pl.ds(g * 32, 16)] = jnp.maximum(lo, 0.0)
                    obp[r, pl.ds(g * 32 + 16, 16)] = jnp.maximum(hi, 0.0)
            pltpu.async_copy(obp, out_hbm.at[chunk0 + c], semo[p])

        issue_idx(0, 0)
        issue_idx(1, 1)
        wait_idx(0)
        issue_gathers(0)

        def pair_body(c2, carry):
            for p in (0, 1):
                c = 2 * c2 + p
                o = 1 - p
                wait_gathers(p)
                issue_idx(c + 2, p)
                wait_idx(o)
                issue_gathers(o)
                @pl.when(c >= 2)
                def _():
                    wait_store(p)
                compute_store(c, p)
            return carry

        lax.fori_loop(0, chunks_per_worker // 2, pair_body, 0)

        wait_gathers(0)
        wait_idx(1)
        wait_store(0)
        wait_store(1)

    return sc_gather


def kernel(pos_start, pos_end, pe, W, b):
    B, S = pos_start.shape
    H = pe.shape[1]
    n_rows = B * S * S

    pe_pad = jnp.pad(pe, ((0, PAD_ROWS - pe.shape[0]), (0, 0)))

    perm = (jnp.arange(H) // 32) * 32 + jnp.where(
        jnp.arange(H) % 2 == 0, (jnp.arange(H) % 32) // 2,
        (jnp.arange(H) % 32) // 2 + 16)
    W_perm = W[perm, :]
    b_perm = b[perm]

    table = pl.pallas_call(
        _table_kernel,
        grid=(NUM_TABLES,),
        in_specs=[
            pl.BlockSpec((PAD_ROWS, H), lambda k: (0, 0)),
            pl.BlockSpec((H, H), lambda k: (0, k)),
            pl.BlockSpec((1, H), lambda k: (0, 0)),
        ],
        out_specs=pl.BlockSpec((1, PAD_ROWS, H), lambda k: (k, 0, 0)),
        out_shape=jax.ShapeDtypeStruct((NUM_TABLES, PAD_ROWS, H),
                                       jnp.bfloat16),
    )(pe_pad, W_perm, b_perm.reshape(1, H))

    idx = pl.pallas_call(
        functools.partial(_idx_kernel, seq=S),
        grid=(NUM_TABLES, B),
        in_specs=[
            pl.BlockSpec((1, 1, S), lambda k, bb: (bb, 0, 0)),
            pl.BlockSpec((1, 1, S), lambda k, bb: (bb, 0, 0)),
        ],
        out_specs=pl.BlockSpec((1, 1, S, S), lambda k, bb: (k, bb, 0, 0)),
        out_shape=jax.ShapeDtypeStruct((NUM_TABLES, B, S, S), jnp.int32),
    )(pos_start.reshape(B, 1, S), pos_end.reshape(B, 1, S))

    info = plsc.get_sparse_core_info()
    num_workers = info.num_cores * info.num_subcores
    sc_gather = _make_sc_gather(n_rows, H, num_workers, info.num_cores)
    table_i32 = jax.lax.bitcast_convert_type(
        table.reshape(NUM_TABLES * PAD_ROWS, H // 2, 2), jnp.int32)

    out = sc_gather(
        table_i32,
        idx.reshape(NUM_TABLES, n_rows // CHUNK, CHUNK),
    )
    return out.reshape(B, S, S, H)


# speedup vs baseline: 1.0674x; 1.0021x over previous
import functools

import jax
import jax.numpy as jnp
from jax import lax
from jax.experimental import pallas as pl
from jax.experimental.pallas import tpu as pltpu
from jax.experimental.pallas import tpu_sc as plsc

MAXLEN = 512
PE_ROWS = 2 * MAXLEN + 1
PAD_ROWS = 1032
NUM_TABLES = 4
CHUNK = 128


def _table_kernel(pe_ref, w_ref, b_ref, out_ref):
    acc = lax.dot_general(
        pe_ref[...], w_ref[...],
        (((1,), (1,)), ((), ())),
        preferred_element_type=jnp.float32,
    )
    out_ref[0] = (acc + 0.25 * b_ref[...]).astype(jnp.bfloat16)


def _idx_kernel(ps_ref, pe_ref, out_ref, *, seq):
    k = pl.program_id(0)
    s = ps_ref[0, 0]
    e = pe_ref[0, 0]
    row = jnp.where(k < 2, s, e)
    col = jnp.where(k % 2 == 0, s, e)
    base = MAXLEN + k * PAD_ROWS
    row2 = lax.broadcast_in_dim(row, (seq, seq), (0,))
    col2 = lax.broadcast_in_dim(col, (seq, seq), (1,))
    out_ref[0, 0] = row2 - col2 + base


def _make_sc_gather(n_rows, hidden, num_workers, num_cores):
    chunks_per_worker = n_rows // (num_workers * CHUNK)
    n_chunks = n_rows // CHUNK
    mesh = plsc.VectorSubcoreMesh(core_axis_name="c", subcore_axis_name="s")

    @functools.partial(
        pl.kernel,
        mesh=mesh,
        compiler_params=pltpu.CompilerParams(use_tc_tiling_on_sc=False),
        out_type=jax.ShapeDtypeStruct((n_chunks, CHUNK, hidden), jnp.float32),
        scratch_types=[
            pltpu.VMEM((NUM_TABLES, CHUNK), jnp.int32),
            pltpu.VMEM((NUM_TABLES, CHUNK), jnp.int32),
            pltpu.VMEM((NUM_TABLES, CHUNK, hidden // 2), jnp.int32),
            pltpu.VMEM((NUM_TABLES, CHUNK, hidden // 2), jnp.int32),
            pltpu.VMEM((CHUNK, hidden), jnp.float32),
            pltpu.VMEM((CHUNK, hidden), jnp.float32),
            pltpu.SemaphoreType.DMA,
            pltpu.SemaphoreType.DMA,
            pltpu.SemaphoreType.DMA,
            pltpu.SemaphoreType.DMA,
            pltpu.SemaphoreType.DMA,
            pltpu.SemaphoreType.DMA,
        ],
    )
    def sc_gather(table_hbm, idx_hbm, out_hbm,
                  ixA, ixB, gbA, gbB, obA, obB,
                  semiA, semiB, semgA, semgB, semoA, semoB):
        wid = lax.axis_index("s") * num_cores + lax.axis_index("c")
        chunk0 = wid * chunks_per_worker
        last = chunks_per_worker - 1

        ix = (ixA, ixB)
        gb = (gbA, gbB)
        ob = (obA, obB)
        semi = (semiA, semiB)
        semg = (semgA, semgB)
        semo = (semoA, semoB)

        def issue_idx(c, p):
            ch = chunk0 + jnp.minimum(c, last)
            pltpu.async_copy(idx_hbm.at[:, ch], ix[p], semi[p])

        def wait_idx(p):
            pltpu.make_async_copy(idx_hbm.at[:, chunk0], ix[p],
                                  semi[p]).wait()

        def issue_gathers(p):
            for k in range(NUM_TABLES):
                pltpu.async_copy(table_hbm.at[ix[p].at[k]], gb[p].at[k],
                                 semg[p])

        def wait_gathers(p):
            for k in range(NUM_TABLES):
                pltpu.make_async_copy(table_hbm.at[ix[p].at[k]], gb[p].at[k],
                                      semg[p]).wait()

        def wait_store(p):
            pltpu.make_async_copy(ob[p], out_hbm.at[chunk0], semo[p]).wait()

        def compute_store(c, p):
            gbp = gb[p]
            obp = ob[p]

            def lo_f32(w):
                return lax.bitcast_convert_type(w << 16, jnp.float32)

            def hi_f32(w):
                return lax.bitcast_convert_type(w & jnp.int32(-65536),
                                                jnp.float32)

            @plsc.parallel_loop(0, CHUNK, unroll=4)
            def row_body(r):
                for g in range(hidden // 32):
                    sl = pl.ds(g * 16, 16)
                    w0 = gbp[0, r, sl]
                    w1 = gbp[1, r, sl]
                    w2 = gbp[2, r, sl]
                    w3 = gbp[3, r, sl]
                    lo = lo_f32(w0) + lo_f32(w1) + lo_f32(w2) + lo_f32(w3)
                    hi = hi_f32(w0) + hi_f32(w1) + hi_f32(w2) + hi_f32(w3)
                    obp[r, pl.ds(g * 32, 16)] = jnp.maximum(lo, 0.0)
                    obp[r, pl.ds(g * 32 + 16, 16)] = jnp.maximum(hi, 0.0)
            pltpu.async_copy(obp, out_hbm.at[chunk0 + c], semo[p])

        issue_idx(0, 0)
        issue_idx(1, 1)
        wait_idx(0)
        issue_gathers(0)

        def pair_body(c2, carry):
            for p in (0, 1):
                c = 2 * c2 + p
                o = 1 - p
                wait_gathers(p)
                issue_idx(c + 2, p)
                wait_idx(o)
                issue_gathers(o)
                @pl.when(c >= 2)
                def _():
                    wait_store(p)
                compute_store(c, p)
            return carry

        lax.fori_loop(0, chunks_per_worker // 2, pair_body, 0)

        wait_gathers(0)
        wait_idx(1)
        wait_store(0)
        wait_store(1)

    return sc_gather


def kernel(pos_start, pos_end, pe, W, b):
    B, S = pos_start.shape
    H = pe.shape[1]
    n_rows = B * S * S

    pe_pad = jnp.pad(pe, ((0, PAD_ROWS - pe.shape[0]), (0, 0)))

    perm = (jnp.arange(H) // 32) * 32 + jnp.where(
        jnp.arange(H) % 2 == 0, (jnp.arange(H) % 32) // 2,
        (jnp.arange(H) % 32) // 2 + 16)
    W_perm = W[perm, :]
    b_perm = b[perm]

    table = pl.pallas_call(
        _table_kernel,
        grid=(NUM_TABLES,),
        in_specs=[
            pl.BlockSpec((PAD_ROWS, H), lambda k: (0, 0)),
            pl.BlockSpec((H, H), lambda k: (0, k)),
            pl.BlockSpec((1, H), lambda k: (0, 0)),
        ],
        out_specs=pl.BlockSpec((1, PAD_ROWS, H), lambda k: (k, 0, 0)),
        out_shape=jax.ShapeDtypeStruct((NUM_TABLES, PAD_ROWS, H),
                                       jnp.bfloat16),
    )(pe_pad, W_perm, b_perm.reshape(1, H))

    idx = pl.pallas_call(
        functools.partial(_idx_kernel, seq=S),
        grid=(NUM_TABLES, B),
        in_specs=[
            pl.BlockSpec((1, 1, S), lambda k, bb: (bb, 0, 0)),
            pl.BlockSpec((1, 1, S), lambda k, bb: (bb, 0, 0)),
        ],
        out_specs=pl.BlockSpec((1, 1, S, S), lambda k, bb: (k, bb, 0, 0)),
        out_shape=jax.ShapeDtypeStruct((NUM_TABLES, B, S, S), jnp.int32),
    )(pos_start.reshape(B, 1, S), pos_end.reshape(B, 1, S))

    info = plsc.get_sparse_core_info()
    num_workers = info.num_cores * info.num_subcores
    sc_gather = _make_sc_gather(n_rows, H, num_workers, info.num_cores)
    table_i32 = jax.lax.bitcast_convert_type(
        table.reshape(NUM_TABLES * PAD_ROWS, H // 2, 2), jnp.int32)

    out = sc_gather(
        table_i32,
        idx.reshape(NUM_TABLES, n_rows // CHUNK, CHUNK),
    )
    return out.reshape(B, S, S, H)
